# Initial kernel scaffold; baseline (speedup 1.0000x reference)
import jax
import jax.numpy as jnp
from jax.experimental import pallas as pl


def kernel(adj_indices, adj_values, features, W1, b1, W2, b2):
    raise NotImplementedError("write your pallas kernel here")


# speedup vs baseline: 4.3602x; 4.3602x over previous
import functools

import jax
import jax.numpy as jnp
from jax import lax
from jax.experimental import pallas as pl
from jax.experimental.pallas import tpu as pltpu
from jax.experimental.pallas import tpu_sc as plsc

N = 10000
E = 320000
D = 128
L = 16
NC, NS = 2, 16
NW = NC * NS
NP = 10240

E_PER_W = E // NW
CHUNK = 80
N_CHUNKS = E_PER_W // CHUNK
ROWS_PER_TILE = NP // NS


def _spmm_body(src_hbm, dst_hbm, vals_hbm, feat_hbm, zeros_hbm, out_hbm,
               src_v, dst_v, vals_v, rows_v, acc_sh):
    cid = lax.axis_index("c")
    sid = lax.axis_index("s")
    wid = sid * NC + cid

    row0 = sid * ROWS_PER_TILE
    pltpu.sync_copy(zeros_hbm.at[pl.ds(row0, ROWS_PER_TILE)],
                    acc_sh.at[pl.ds(row0, ROWS_PER_TILE)])
    plsc.subcore_barrier()

    wstart = wid * E_PER_W

    def chunk_step(g, _):
        base = wstart + g * CHUNK
        pltpu.sync_copy(src_hbm.at[pl.ds(base, CHUNK)], src_v)
        pltpu.sync_copy(dst_hbm.at[pl.ds(base, CHUNK)], dst_v)
        pltpu.sync_copy(vals_hbm.at[pl.ds(base, CHUNK)], vals_v)
        pltpu.sync_copy(feat_hbm.at[src_v], rows_v)

        def scale_block(b, _):
            ve = vals_v[pl.ds(b * L, L)]
            for j in range(L):
                e = b * L + j
                vj = jnp.full((L,), ve[j], dtype=jnp.float32)
                for k in range(D // L):
                    rows_v[e, pl.ds(k * L, L)] = rows_v[e, pl.ds(k * L, L)] * vj
            return 0

        lax.fori_loop(0, CHUNK // L, scale_block, 0)

        pltpu.sync_copy(rows_v, acc_sh.at[dst_v], add=True)
        return 0

    lax.fori_loop(0, N_CHUNKS, chunk_step, 0)

    plsc.subcore_barrier()
    pltpu.sync_copy(acc_sh.at[pl.ds(row0, ROWS_PER_TILE)],
                    out_hbm.at[cid, pl.ds(row0, ROWS_PER_TILE)])


_spmm = functools.partial(
    pl.kernel,
    out_type=jax.ShapeDtypeStruct((NC, NP, D), jnp.float32),
    mesh=plsc.VectorSubcoreMesh(core_axis_name="c", subcore_axis_name="s",
                                num_cores=NC, num_subcores=NS),
    scratch_types=[
        pltpu.VMEM((CHUNK,), jnp.int32),
        pltpu.VMEM((CHUNK,), jnp.int32),
        pltpu.VMEM((CHUNK,), jnp.float32),
        pltpu.VMEM((CHUNK, D), jnp.float32),
        pltpu.VMEM_SHARED((NP, D), jnp.float32),
    ],
)(_spmm_body)


def _mlp_body(f_ref, p0_ref, p1_ref, w1t_ref, b1_ref, w2t_ref, b2_ref, o_ref):
    x = f_ref[...] + p0_ref[...] + p1_ref[...]
    h = jnp.maximum(
        jnp.dot(x, w1t_ref[...], preferred_element_type=jnp.float32)
        + b1_ref[...], 0.0)
    o_ref[...] = (jnp.dot(h, w2t_ref[...], preferred_element_type=jnp.float32)
                  + b2_ref[...])


BLK = 2048


def _mlp(f_pad, p0, p1, w1t, b1, w2t, b2):
    grid = (NP // BLK,)
    row_spec = pl.BlockSpec((BLK, D), lambda i: (i, 0))
    full_spec = pl.BlockSpec((D, D), lambda i: (0, 0))
    bias_spec = pl.BlockSpec((1, D), lambda i: (0, 0))
    return pl.pallas_call(
        _mlp_body,
        grid=grid,
        in_specs=[row_spec, row_spec, row_spec,
                  full_spec, bias_spec, full_spec, bias_spec],
        out_specs=row_spec,
        out_shape=jax.ShapeDtypeStruct((NP, D), jnp.float32),
    )(f_pad, p0, p1, w1t, b1, w2t, b2)


def kernel(adj_indices, adj_values, features, W1, b1, W2, b2):
    dst = adj_indices[0]
    src = adj_indices[1]
    f_pad = jnp.pad(features, ((0, NP - N), (0, 0)))
    zeros = jnp.zeros((NP, D), jnp.float32)
    partials = _spmm(src, dst, adj_values, f_pad, zeros)
    out = _mlp(f_pad, partials[0], partials[1],
               W1.T, b1.reshape(1, D), W2.T, b2.reshape(1, D))
    return out[:N]


# speedup vs baseline: 9.4343x; 2.1637x over previous
import functools

import jax
import jax.numpy as jnp
from jax import lax
from jax.experimental import pallas as pl
from jax.experimental.pallas import tpu as pltpu
from jax.experimental.pallas import tpu_sc as plsc

N = 10000
E = 320000
D = 128
L = 16
NC, NS = 2, 16
NW = NC * NS
NP = 10112

E_PER_W = E // NW
CHUNK = 80
N_CHUNKS = E_PER_W // CHUNK
ROWS_PER_TILE = NP // NS


def _spmm_body(src_hbm, dst_hbm, vals_hbm, feat_hbm, zeros_hbm, out_hbm,
               src_all, vals_all, dst0, dst1, rows0, rows1,
               dsem0, dsem1, gsem0, gsem1, ssem0, ssem1, acc_sh):
    cid = lax.axis_index("c")
    sid = lax.axis_index("s")
    wid = sid * NC + cid

    row0 = sid * ROWS_PER_TILE
    pltpu.sync_copy(zeros_hbm.at[pl.ds(row0, ROWS_PER_TILE)],
                    acc_sh.at[pl.ds(row0, ROWS_PER_TILE)])
    e0 = wid * E_PER_W
    pltpu.sync_copy(src_hbm.at[pl.ds(e0, E_PER_W)], src_all)
    pltpu.sync_copy(vals_hbm.at[pl.ds(e0, E_PER_W)], vals_all)
    plsc.subcore_barrier()

    rows = (rows0, rows1)
    dst = (dst0, dst1)
    dsem = (dsem0, dsem1)
    gsem = (gsem0, gsem1)
    ssem = (ssem0, ssem1)

    def scale(g, buf):
        def blk(b, _):
            ve = vals_all[pl.ds(g * CHUNK + b * L, L)]
            for j in range(L):
                e = b * L + j
                vj = jnp.full((L,), ve[j], dtype=jnp.float32)
                for k in range(D // L):
                    buf[e, pl.ds(k * L, L)] = buf[e, pl.ds(k * L, L)] * vj
            return 0
        lax.fori_loop(0, CHUNK // L, blk, 0)

    def issue_dst(g, b):
        pltpu.async_copy(dst_hbm.at[pl.ds(e0 + g * CHUNK, CHUNK)],
                         dst[b], dsem[b])

    def issue_gather(g, b):
        pltpu.async_copy(feat_hbm.at[src_all.at[pl.ds(g * CHUNK, CHUNK)]],
                         rows[b], gsem[b])

    issue_dst(0, 0)
    issue_gather(0, 0)

    def step(g, cur, nxt, last):
        pltpu.make_async_copy(feat_hbm.at[src_all.at[pl.ds(0, CHUNK)]],
                              rows[cur], gsem[cur]).wait()

        @pl.when(g >= 1)
        def _():
            pltpu.make_async_copy(rows[nxt], acc_sh.at[dst[nxt]],
                                  ssem[nxt]).wait()

        if not last:
            issue_dst(g + 1, nxt)
            issue_gather(g + 1, nxt)
        scale(g, rows[cur])
        pltpu.make_async_copy(dst_hbm.at[pl.ds(e0, CHUNK)], dst[cur],
                              dsem[cur]).wait()
        if last:
            pltpu.sync_copy(rows[cur], acc_sh.at[dst[cur]], add=True)
        else:
            pltpu.async_copy(rows[cur], acc_sh.at[dst[cur]], ssem[cur],
                             add=True)

    def outer(i, _):
        for b in range(2):
            step(i * 2 + b, b, 1 - b, False)
        return 0

    lax.fori_loop(0, (N_CHUNKS - 1) // 2, outer, 0)
    step(N_CHUNKS - 1, 0, 1, True)

    plsc.subcore_barrier()
    pltpu.sync_copy(acc_sh.at[pl.ds(row0, ROWS_PER_TILE)],
                    out_hbm.at[cid, pl.ds(row0, ROWS_PER_TILE)])


_spmm = functools.partial(
    pl.kernel,
    out_type=jax.ShapeDtypeStruct((NC, NP, D), jnp.float32),
    mesh=plsc.VectorSubcoreMesh(core_axis_name="c", subcore_axis_name="s",
                                num_cores=NC, num_subcores=NS),
    scratch_types=[
        pltpu.VMEM((E_PER_W,), jnp.int32),
        pltpu.VMEM((E_PER_W,), jnp.float32),
        pltpu.VMEM((CHUNK,), jnp.int32),
        pltpu.VMEM((CHUNK,), jnp.int32),
        pltpu.VMEM((CHUNK, D), jnp.float32),
        pltpu.VMEM((CHUNK, D), jnp.float32),
        pltpu.SemaphoreType.DMA,
        pltpu.SemaphoreType.DMA,
        pltpu.SemaphoreType.DMA,
        pltpu.SemaphoreType.DMA,
        pltpu.SemaphoreType.DMA,
        pltpu.SemaphoreType.DMA,
        pltpu.VMEM_SHARED((NP, D), jnp.float32),
    ],
)(_spmm_body)


def _mlp_body(f_ref, p0_ref, p1_ref, w1t_ref, b1_ref, w2t_ref, b2_ref, o_ref):
    x = f_ref[...] + p0_ref[...] + p1_ref[...]
    h = jnp.maximum(
        jnp.dot(x, w1t_ref[...], preferred_element_type=jnp.float32)
        + b1_ref[...], 0.0)
    o_ref[...] = (jnp.dot(h, w2t_ref[...], preferred_element_type=jnp.float32)
                  + b2_ref[...])


BLK = 1264


def _mlp(f_pad, p0, p1, w1t, b1, w2t, b2):
    grid = (NP // BLK,)
    row_spec = pl.BlockSpec((BLK, D), lambda i: (i, 0))
    full_spec = pl.BlockSpec((D, D), lambda i: (0, 0))
    bias_spec = pl.BlockSpec((1, D), lambda i: (0, 0))
    return pl.pallas_call(
        _mlp_body,
        grid=grid,
        in_specs=[row_spec, row_spec, row_spec,
                  full_spec, bias_spec, full_spec, bias_spec],
        out_specs=row_spec,
        out_shape=jax.ShapeDtypeStruct((NP, D), jnp.float32),
    )(f_pad, p0, p1, w1t, b1, w2t, b2)


def kernel(adj_indices, adj_values, features, W1, b1, W2, b2):
    dst = adj_indices[0]
    src = adj_indices[1]
    f_pad = jnp.pad(features, ((0, NP - N), (0, 0)))
    zeros = jnp.zeros((NP, D), jnp.float32)
    partials = _spmm(src, dst, adj_values, f_pad, zeros)
    out = _mlp(f_pad, partials[0], partials[1],
               W1.T, b1.reshape(1, D), W2.T, b2.reshape(1, D))
    return out[:N]


# speedup vs baseline: 9.8966x; 1.0490x over previous
import functools

import jax
import jax.numpy as jnp
from jax import lax
from jax.experimental import pallas as pl
from jax.experimental.pallas import tpu as pltpu
from jax.experimental.pallas import tpu_sc as plsc

N = 10000
E = 320000
D = 128
L = 16
NC, NS = 2, 16
NW = NC * NS
NP = 10112

E_PER_W = E // NW
CHUNK = 80
N_CHUNKS = E_PER_W // CHUNK
ROWS_PER_TILE = NP // NS


def _spmm_body(src_hbm, dst_hbm, vals_hbm, feat_hbm, out_hbm,
               src_all, vals_all, dst0, dst1, rows0, rows1,
               dsem0, dsem1, gsem0, gsem1, ssem0, ssem1, acc_sh):
    cid = lax.axis_index("c")
    sid = lax.axis_index("s")
    wid = sid * NC + cid

    zvec = jnp.zeros((L,), jnp.float32)

    def zblk(e, _):
        for k in range(D // L):
            rows0[e, pl.ds(k * L, L)] = zvec
        return 0

    lax.fori_loop(0, CHUNK, zblk, 0)
    row0 = sid * ROWS_PER_TILE
    for i in range(7):
        pltpu.sync_copy(rows0, acc_sh.at[pl.ds(row0 + i * CHUNK, CHUNK)])
    pltpu.sync_copy(rows0.at[pl.ds(0, ROWS_PER_TILE - 7 * CHUNK)],
                    acc_sh.at[pl.ds(row0 + 7 * CHUNK,
                                    ROWS_PER_TILE - 7 * CHUNK)])
    e0 = wid * E_PER_W
    pltpu.sync_copy(src_hbm.at[pl.ds(e0, E_PER_W)], src_all)
    pltpu.sync_copy(vals_hbm.at[pl.ds(e0, E_PER_W)], vals_all)
    plsc.subcore_barrier()

    rows = (rows0, rows1)
    dst = (dst0, dst1)
    dsem = (dsem0, dsem1)
    gsem = (gsem0, gsem1)
    ssem = (ssem0, ssem1)

    def scale(g, buf):
        def blk(b, _):
            ve = vals_all[pl.ds(g * CHUNK + b * L, L)]
            for j in range(L):
                e = b * L + j
                vj = jnp.full((L,), ve[j], dtype=jnp.float32)
                for k in range(D // L):
                    buf[e, pl.ds(k * L, L)] = buf[e, pl.ds(k * L, L)] * vj
            return 0
        lax.fori_loop(0, CHUNK // L, blk, 0)

    def issue_dst(g, b):
        pltpu.async_copy(dst_hbm.at[pl.ds(e0 + g * CHUNK, CHUNK)],
                         dst[b], dsem[b])

    def issue_gather(g, b):
        pltpu.async_copy(feat_hbm.at[src_all.at[pl.ds(g * CHUNK, CHUNK)]],
                         rows[b], gsem[b])

    issue_dst(0, 0)
    issue_gather(0, 0)

    def step(g, cur, nxt, last):
        pltpu.make_async_copy(feat_hbm.at[src_all.at[pl.ds(0, CHUNK)]],
                              rows[cur], gsem[cur]).wait()

        @pl.when(g >= 1)
        def _():
            pltpu.make_async_copy(rows[nxt], acc_sh.at[dst[nxt]],
                                  ssem[nxt]).wait()

        if not last:
            issue_dst(g + 1, nxt)
            issue_gather(g + 1, nxt)
        scale(g, rows[cur])
        pltpu.make_async_copy(dst_hbm.at[pl.ds(e0, CHUNK)], dst[cur],
                              dsem[cur]).wait()
        if last:
            pltpu.sync_copy(rows[cur], acc_sh.at[dst[cur]], add=True)
        else:
            pltpu.async_copy(rows[cur], acc_sh.at[dst[cur]], ssem[cur],
                             add=True)

    def outer(i, _):
        for b in range(2):
            step(i * 2 + b, b, 1 - b, False)
        return 0

    lax.fori_loop(0, (N_CHUNKS - 1) // 2, outer, 0)
    step(N_CHUNKS - 1, 0, 1, True)

    plsc.subcore_barrier()
    pltpu.sync_copy(acc_sh.at[pl.ds(row0, ROWS_PER_TILE)],
                    out_hbm.at[cid, pl.ds(row0, ROWS_PER_TILE)])


_spmm = functools.partial(
    pl.kernel,
    out_type=jax.ShapeDtypeStruct((NC, NP, D), jnp.float32),
    mesh=plsc.VectorSubcoreMesh(core_axis_name="c", subcore_axis_name="s",
                                num_cores=NC, num_subcores=NS),
    scratch_types=[
        pltpu.VMEM((E_PER_W,), jnp.int32),
        pltpu.VMEM((E_PER_W,), jnp.float32),
        pltpu.VMEM((CHUNK,), jnp.int32),
        pltpu.VMEM((CHUNK,), jnp.int32),
        pltpu.VMEM((CHUNK, D), jnp.float32),
        pltpu.VMEM((CHUNK, D), jnp.float32),
        pltpu.SemaphoreType.DMA,
        pltpu.SemaphoreType.DMA,
        pltpu.SemaphoreType.DMA,
        pltpu.SemaphoreType.DMA,
        pltpu.SemaphoreType.DMA,
        pltpu.SemaphoreType.DMA,
        pltpu.VMEM_SHARED((NP, D), jnp.float32),
    ],
)(_spmm_body)


def _mlp_body(f_ref, p0_ref, p1_ref, w1t_ref, b1_ref, w2t_ref, b2_ref, o_ref):
    x = f_ref[...] + p0_ref[...] + p1_ref[...]
    h = jnp.maximum(
        jnp.dot(x, w1t_ref[...], preferred_element_type=jnp.float32)
        + b1_ref[...], 0.0)
    o_ref[...] = (jnp.dot(h, w2t_ref[...], preferred_element_type=jnp.float32)
                  + b2_ref[...])


BLK = 1000


def _mlp(feats, p0, p1, w1t, b1, w2t, b2):
    grid = (N // BLK,)
    row_spec = pl.BlockSpec((BLK, D), lambda i: (i, 0))
    full_spec = pl.BlockSpec((D, D), lambda i: (0, 0))
    bias_spec = pl.BlockSpec((1, D), lambda i: (0, 0))
    return pl.pallas_call(
        _mlp_body,
        grid=grid,
        in_specs=[row_spec, row_spec, row_spec,
                  full_spec, bias_spec, full_spec, bias_spec],
        out_specs=row_spec,
        out_shape=jax.ShapeDtypeStruct((N, D), jnp.float32),
    )(feats, p0, p1, w1t, b1, w2t, b2)


def kernel(adj_indices, adj_values, features, W1, b1, W2, b2):
    dst = adj_indices[0]
    src = adj_indices[1]
    partials = _spmm(src, dst, adj_values, features)
    return _mlp(features, partials[0], partials[1],
                W1.T, b1.reshape(1, D), W2.T, b2.reshape(1, D))
